# Initial kernel scaffold; baseline (speedup 1.0000x reference)
import jax
import jax.numpy as jnp
from jax.experimental import pallas as pl


def kernel(x, edge_indices, batch, g0_Wl, g0_bl, g0_Wr, g0_br, g0_att, g0_bias, g1_Wl, g1_bl, g1_Wr, g1_br, g1_att, g1_bias, dp0_W, dp0_b, dp1_W, dp1_b, fin_W, fin_b):
    raise NotImplementedError("write your pallas kernel here")


# speedup vs baseline: 1.0186x; 1.0186x over previous
import functools

import jax
import jax.numpy as jnp
from jax.experimental import pallas as pl

N = 10000
B = 100
D = 128
DOUT = 64
ROWS_PER_BLOCK = 1000
NUM_BLOCKS = N // ROWS_PER_BLOCK


def _gat_layer(x, src, dst, Wl, bl, Wr, br, att, bias):
    xl = x @ Wl.T + bl
    xr = x @ Wr.T + br
    e = jax.nn.leaky_relu(xl[src] + xr[dst], 0.2) @ att
    emax = jax.ops.segment_max(e, dst, num_segments=N)
    p = jnp.exp(e - emax[dst])
    den = jax.ops.segment_sum(p, dst, num_segments=N)
    alpha = p / (den[dst] + 1e-16)
    return jax.ops.segment_sum(alpha[:, None] * xl[src], dst, num_segments=N) + bias


def _norm(x, batch):
    cnt = jnp.clip(jnp.bincount(batch, length=B).astype(x.dtype), 1.0)
    mean = jax.ops.segment_sum(x, batch, num_segments=B) / cnt[:, None]
    xc = x - mean[batch]
    var = jax.ops.segment_sum(xc * xc, batch, num_segments=B) / cnt[:, None]
    return xc / jnp.sqrt(var[batch] + 1e-5)


def _head_kernel(x_ref, batch_ref, w_ref, b_ref, out_ref, acc_ref):
    i = pl.program_id(0)

    @pl.when(i == 0)
    def _init():
        acc_ref[...] = jnp.zeros_like(acc_ref)

    rows = x_ref[...]
    bvals = batch_ref[0, 0, :]
    onehot = (bvals[:, None] == jax.lax.broadcasted_iota(jnp.int32, (ROWS_PER_BLOCK, B), 1)).astype(jnp.float32)
    acc_ref[...] += jax.lax.dot_general(
        onehot, rows, (((0,), (0,)), ((), ())), preferred_element_type=jnp.float32)

    @pl.when(i == NUM_BLOCKS - 1)
    def _fin():
        out_ref[...] = jax.lax.dot_general(
            acc_ref[...], w_ref[...], (((1,), (1,)), ((), ())),
            preferred_element_type=jnp.float32) + b_ref[...]


def _pooled_head(x_g, batch, fin_W, fin_b):
    batch2 = batch.reshape(NUM_BLOCKS, 1, ROWS_PER_BLOCK)
    return pl.pallas_call(
        _head_kernel,
        grid=(NUM_BLOCKS,),
        in_specs=[
            pl.BlockSpec((ROWS_PER_BLOCK, D), lambda i: (i, 0)),
            pl.BlockSpec((1, 1, ROWS_PER_BLOCK), lambda i: (i, 0, 0)),
            pl.BlockSpec((DOUT, D), lambda i: (0, 0)),
            pl.BlockSpec((1, DOUT), lambda i: (0, 0)),
        ],
        out_specs=pl.BlockSpec((B, DOUT), lambda i: (0, 0)),
        out_shape=jax.ShapeDtypeStruct((B, DOUT), jnp.float32),
        scratch_shapes=[pltpu_vmem((B, D), jnp.float32)],
    )(x_g, batch2, fin_W, fin_b.reshape(1, DOUT))


def pltpu_vmem(shape, dtype):
    from jax.experimental.pallas import tpu as pltpu
    return pltpu.VMEM(shape, dtype)


def kernel(x, edge_indices, batch, g0_Wl, g0_bl, g0_Wr, g0_br, g0_att, g0_bias,
           g1_Wl, g1_bl, g1_Wr, g1_br, g1_att, g1_bias,
           dp0_W, dp0_b, dp1_W, dp1_b, fin_W, fin_b):
    n = x.shape[0]
    loop = jnp.arange(n, dtype=jnp.int32)
    ei = edge_indices[0]
    src = jnp.concatenate([ei[0], loop])
    dst = jnp.concatenate([ei[1], loop])

    x_g = x
    for (Wl, bl, Wr, br, att, bias) in (
            (g0_Wl, g0_bl, g0_Wr, g0_br, g0_att, g0_bias),
            (g1_Wl, g1_bl, g1_Wr, g1_br, g1_att, g1_bias)):
        h = _gat_layer(x_g, src, dst, Wl, bl, Wr, br, att, bias)
        h = _norm(h, batch)
        h = jax.nn.relu(h)
        h = h + x_g
        x_g = h

    return _pooled_head(x_g, batch, fin_W, fin_b)


# speedup vs baseline: 6.3146x; 6.1990x over previous
import functools

import jax
import jax.numpy as jnp
from jax import lax
from jax.experimental import pallas as pl
from jax.experimental.pallas import tpu as pltpu
from jax.experimental.pallas import tpu_sc as plsc

_USE_JNP_EDGE = False
N = 10000
B = 100
D = 128
DOUT = 64
NT = 32
NPT = 320
NP = NT * NPT
E2 = 160000 + N
CE = 64
E2P = ((E2 + CE - 1) // CE) * CE
EPAD = E2P - E2
EBUF = 7168
NL = 16
RB = 1024
NB = NP // RB
HB = 1000
NHB = N // HB


def _splat_i(v):
    return jnp.full((NL,), v, jnp.int32)


def _sread(ref, i):
    return ref[pl.ds(i, 16)][0]


def _sc_body(xl_hbm, xr_hbm, srcs_hbm, dsts_hbm, tb_hbm, att_hbm, m_hbm,
             out_hbm, out_loc, den_loc, rows_v, rowsr_v, idx_v, dst_v, tb_v,
             tb_s, dst_s, att_loc, m_loc, sem0, sem1, sem2, sem3):
    cid = lax.axis_index("c")
    sid = lax.axis_index("s")
    wid = sid * 2 + cid
    n0 = wid * NPT

    pltpu.sync_copy(tb_hbm, tb_v)
    pltpu.sync_copy(att_hbm, att_loc)
    pltpu.sync_copy(m_hbm, m_loc)
    att8 = [att_loc[pl.ds(16 * k, 16)] for k in range(8)]
    m16 = m_loc[pl.ds(0, 16)]
    sems = [sem0, sem1]
    semsr = [sem2, sem3]

    for g in range(4):
        v = tb_v[pl.ds(16 * g, 16)]
        for l in range(16):
            tb_s[16 * g + l] = v[l]
    e0 = tb_s[wid]
    e1 = tb_s[wid + 1]
    c0 = e0 // CE
    base = c0 * CE
    nch = (e1 - base + CE - 1) // CE
    ng = (nch + 1) // 2
    GRP = CE // 16

    zero16 = jnp.zeros((16,), jnp.float32)

    def _zrow(i, _):
        for k in range(8):
            out_loc[i, pl.ds(16 * k, 16)] = zero16
        den_loc[i, pl.ds(0, 16)] = zero16
        return 0
    lax.fori_loop(0, NPT, _zrow, 0)

    def fetch_idx(t, b):
        off = base + t * CE
        pltpu.sync_copy(srcs_hbm.at[pl.ds(off, CE)], idx_v.at[b])
        pltpu.sync_copy(dsts_hbm.at[pl.ds(off, CE)], dst_v.at[b])

    def start_gather(b):
        pltpu.async_copy(xl_hbm.at[idx_v.at[b]], rows_v.at[b], sems[b])
        pltpu.async_copy(xr_hbm.at[dst_v.at[b]], rowsr_v.at[b], semsr[b])

    def wait_gather(b):
        pltpu.make_async_copy(xl_hbm.at[idx_v.at[b]], rows_v.at[b],
                              sems[b]).wait()
        pltpu.make_async_copy(xr_hbm.at[dst_v.at[b]], rowsr_v.at[b],
                              semsr[b]).wait()

    fetch_idx(0, 0)
    start_gather(0)

    def edge_body(cbase, b):
        def body(j, carry):
            jj = j - cbase
            dl = dst_s[b, jj] - n0
            acc = zero16
            rowk = []
            for k in range(8):
                a = rows_v[b, jj, pl.ds(16 * k, 16)]
                rowk.append(a)
                r = rowsr_v[b, jj, pl.ds(16 * k, 16)]
                s = a + r
                lr = jnp.maximum(s, 0.2 * s)
                acc = acc + lr * att8[k]
            e = jnp.sum(acc)
            p = jnp.exp(jnp.full((16,), e) - m16)
            den_loc[dl, pl.ds(0, 16)] = den_loc[dl, pl.ds(0, 16)] + p
            for k in range(8):
                out_loc[dl, pl.ds(16 * k, 16)] = (
                    out_loc[dl, pl.ds(16 * k, 16)] + p * rowk[k])
            return carry
        return body

    def group(g, carry):
        for b in range(2):
            t = g * 2 + b

            @pl.when(t < nch)
            def _():
                wait_gather(b)

                @pl.when(t + 1 < nch)
                def _():
                    fetch_idx(t + 1, 1 - b)
                    start_gather(1 - b)

                for q in range(CE // 16):
                    dl16 = dst_v[b, pl.ds(16 * q, 16)]
                    for l in range(16):
                        dst_s[b, 16 * q + l] = dl16[l]

            cbase = base + t * CE
            lo = jnp.maximum(e0, cbase)
            hi = jnp.minimum(e1, cbase + CE)
            carry = lax.fori_loop(lo, hi, edge_body(cbase, b), carry)
        return carry

    lax.fori_loop(0, ng, group, jnp.int32(0))

    def _nrow(i, _):
        denv = den_loc[i, pl.ds(0, 16)]
        rcp = jnp.where(denv > 0.0, 1.0 / denv, 0.0)
        for k in range(8):
            out_loc[i, pl.ds(16 * k, 16)] = out_loc[i, pl.ds(16 * k, 16)] * rcp
        return 0
    lax.fori_loop(0, NPT, _nrow, 0)

    pltpu.sync_copy(out_loc, out_hbm.at[pl.ds(n0, NPT)])


_sc_gat = functools.partial(
    pl.kernel,
    out_type=jax.ShapeDtypeStruct((NP, D), jnp.float32),
    mesh=plsc.VectorSubcoreMesh(core_axis_name="c", subcore_axis_name="s"),
    compiler_params=pltpu.CompilerParams(needs_layout_passes=False),
    scratch_types=[
        pltpu.VMEM((NPT, D), jnp.float32),
        pltpu.VMEM((NPT, 16), jnp.float32),
        pltpu.VMEM((2, CE, D), jnp.float32),
        pltpu.VMEM((2, CE, D), jnp.float32),
        pltpu.VMEM((2, CE), jnp.int32),
        pltpu.VMEM((2, CE), jnp.int32),
        pltpu.VMEM((64,), jnp.int32),
        pltpu.SMEM((64,), jnp.int32),
        pltpu.SMEM((2, CE), jnp.int32),
        pltpu.VMEM((D,), jnp.float32),
        pltpu.VMEM((16,), jnp.float32),
        pltpu.SemaphoreType.DMA,
        pltpu.SemaphoreType.DMA,
        pltpu.SemaphoreType.DMA,
        pltpu.SemaphoreType.DMA,
    ])(_sc_body)


def _mm2_kernel(x_ref, wl_ref, bl_ref, wr_ref, br_ref, xl_ref, xr_ref):
    xb = x_ref[...]
    xl_ref[...] = lax.dot_general(
        xb, wl_ref[...], (((1,), (1,)), ((), ())),
        preferred_element_type=jnp.float32) + bl_ref[...]
    xr_ref[...] = lax.dot_general(
        xb, wr_ref[...], (((1,), (1,)), ((), ())),
        preferred_element_type=jnp.float32) + br_ref[...]


def _mm2(x, Wl, bl, Wr, br):
    return pl.pallas_call(
        _mm2_kernel, grid=(NB,),
        in_specs=[
            pl.BlockSpec((RB, D), lambda i: (i, 0)),
            pl.BlockSpec((D, D), lambda i: (0, 0)),
            pl.BlockSpec((1, D), lambda i: (0, 0)),
            pl.BlockSpec((D, D), lambda i: (0, 0)),
            pl.BlockSpec((1, D), lambda i: (0, 0)),
        ],
        out_specs=[pl.BlockSpec((RB, D), lambda i: (i, 0))] * 2,
        out_shape=[jax.ShapeDtypeStruct((NP, D), jnp.float32)] * 2,
    )(x, Wl, bl.reshape(1, D), Wr, br.reshape(1, D))


def _stats_kernel(h_ref, batch_ref, sums_ref, sqs_ref, cnt_ref,
                  acc_s, acc_q, acc_c):
    i = pl.program_id(0)

    @pl.when(i == 0)
    def _():
        acc_s[...] = jnp.zeros_like(acc_s)
        acc_q[...] = jnp.zeros_like(acc_q)
        acc_c[...] = jnp.zeros_like(acc_c)

    hb = h_ref[...]
    bv = batch_ref[0, 0, :]
    oh = (bv[:, None] ==
          lax.broadcasted_iota(jnp.int32, (RB, B), 1)).astype(jnp.float32)
    acc_s[...] += lax.dot_general(oh, hb, (((0,), (0,)), ((), ())),
                                  preferred_element_type=jnp.float32)
    acc_q[...] += lax.dot_general(oh, hb * hb, (((0,), (0,)), ((), ())),
                                  preferred_element_type=jnp.float32)
    acc_c[...] += jnp.sum(oh, axis=0)[None, :]

    @pl.when(i == NB - 1)
    def _():
        sums_ref[...] = acc_s[...]
        sqs_ref[...] = acc_q[...]
        cnt_ref[...] = acc_c[...]


def _stats(h, batch3):
    return pl.pallas_call(
        _stats_kernel, grid=(NB,),
        in_specs=[
            pl.BlockSpec((RB, D), lambda i: (i, 0)),
            pl.BlockSpec((1, 1, RB), lambda i: (i, 0, 0)),
        ],
        out_specs=[
            pl.BlockSpec((B, D), lambda i: (0, 0)),
            pl.BlockSpec((B, D), lambda i: (0, 0)),
            pl.BlockSpec((1, B), lambda i: (0, 0)),
        ],
        out_shape=[
            jax.ShapeDtypeStruct((B, D), jnp.float32),
            jax.ShapeDtypeStruct((B, D), jnp.float32),
            jax.ShapeDtypeStruct((1, B), jnp.float32),
        ],
        scratch_shapes=[
            pltpu.VMEM((B, D), jnp.float32),
            pltpu.VMEM((B, D), jnp.float32),
            pltpu.VMEM((1, B), jnp.float32),
        ],
    )(h, batch3)


def _apply_kernel(h_ref, xp_ref, batch_ref, sums_ref, sqs_ref, cnt_ref,
                  out_ref):
    bv = batch_ref[0, 0, :]
    oh = (bv[:, None] ==
          lax.broadcasted_iota(jnp.int32, (RB, B), 1)).astype(jnp.float32)
    cnt = jnp.maximum(cnt_ref[0, :], 1.0)
    mean = sums_ref[...] / cnt[:, None]
    var = sqs_ref[...] / cnt[:, None] - mean * mean
    meanrow = lax.dot_general(oh, mean, (((1,), (0,)), ((), ())),
                              preferred_element_type=jnp.float32)
    varrow = lax.dot_general(oh, var, (((1,), (0,)), ((), ())),
                             preferred_element_type=jnp.float32)
    xc = h_ref[...] - meanrow
    hn = xc * lax.rsqrt(varrow + 1e-5)
    out_ref[...] = jnp.where(bv[:, None] < B,
                             jnp.maximum(hn, 0.0) + xp_ref[...], 0.0)


def _apply(h, xprev, batch3, sums, sqs, cnt):
    return pl.pallas_call(
        _apply_kernel, grid=(NB,),
        in_specs=[
            pl.BlockSpec((RB, D), lambda i: (i, 0)),
            pl.BlockSpec((RB, D), lambda i: (i, 0)),
            pl.BlockSpec((1, 1, RB), lambda i: (i, 0, 0)),
            pl.BlockSpec((B, D), lambda i: (0, 0)),
            pl.BlockSpec((B, D), lambda i: (0, 0)),
            pl.BlockSpec((1, B), lambda i: (0, 0)),
        ],
        out_specs=pl.BlockSpec((RB, D), lambda i: (i, 0)),
        out_shape=jax.ShapeDtypeStruct((NP, D), jnp.float32),
    )(h, xprev, batch3, sums, sqs, cnt)


def _head_kernel(x_ref, batch_ref, w_ref, b_ref, out_ref, acc_ref):
    i = pl.program_id(0)

    @pl.when(i == 0)
    def _():
        acc_ref[...] = jnp.zeros_like(acc_ref)

    rows = x_ref[...]
    bvals = batch_ref[0, 0, :]
    onehot = (bvals[:, None] ==
              lax.broadcasted_iota(jnp.int32, (HB, B), 1)).astype(jnp.float32)
    acc_ref[...] += lax.dot_general(onehot, rows, (((0,), (0,)), ((), ())),
                                    preferred_element_type=jnp.float32)

    @pl.when(i == NHB - 1)
    def _():
        out_ref[...] = lax.dot_general(
            acc_ref[...], w_ref[...], (((1,), (1,)), ((), ())),
            preferred_element_type=jnp.float32) + b_ref[...]


def _pooled_head(x_g, batch, fin_W, fin_b):
    batch3 = batch.reshape(NHB, 1, HB)
    return pl.pallas_call(
        _head_kernel,
        grid=(NHB,),
        in_specs=[
            pl.BlockSpec((HB, D), lambda i: (i, 0)),
            pl.BlockSpec((1, 1, HB), lambda i: (i, 0, 0)),
            pl.BlockSpec((DOUT, D), lambda i: (0, 0)),
            pl.BlockSpec((1, DOUT), lambda i: (0, 0)),
        ],
        out_specs=pl.BlockSpec((B, DOUT), lambda i: (0, 0)),
        out_shape=jax.ShapeDtypeStruct((B, DOUT), jnp.float32),
        scratch_shapes=[pltpu.VMEM((B, D), jnp.float32)],
    )(x_g, batch3, fin_W, fin_b.reshape(1, DOUT))


def kernel(x, edge_indices, batch, g0_Wl, g0_bl, g0_Wr, g0_br, g0_att,
           g0_bias, g1_Wl, g1_bl, g1_Wr, g1_br, g1_att, g1_bias,
           dp0_W, dp0_b, dp1_W, dp1_b, fin_W, fin_b):
    x_pad = jnp.concatenate(
        [x, jnp.zeros((NP - N, D), jnp.float32)], axis=0)
    batch_pad = jnp.concatenate(
        [batch.astype(jnp.int32), jnp.full((NP - N,), B, jnp.int32)])
    batch3 = batch_pad.reshape(NB, 1, RB)

    loop = jnp.arange(N, dtype=jnp.int32)
    ei = edge_indices[0]
    srcp = jnp.concatenate([ei[0].astype(jnp.int32), loop,
                            jnp.zeros((EPAD,), jnp.int32)])
    dstp = jnp.concatenate([ei[1].astype(jnp.int32), loop,
                            jnp.full((EPAD,), NP - 1, jnp.int32)])
    order = jnp.argsort(dstp)
    srcs = srcp[order]
    dsts = dstp[order]
    tb = jnp.searchsorted(
        dsts, jnp.arange(NT + 1, dtype=jnp.int32) * NPT).astype(jnp.int32)
    tb = jnp.concatenate([tb, jnp.zeros((64 - NT - 1,), jnp.int32)])

    x_g = x_pad
    for (Wl, bl, Wr, br, att) in (
            (g0_Wl, g0_bl, g0_Wr, g0_br, g0_att),
            (g1_Wl, g1_bl, g1_Wr, g1_br, g1_att)):
        xl, xr = _mm2(x_g, Wl, bl, Wr, br)
        mshift = jnp.sqrt(jnp.dot(att, att)) * (
            jnp.sqrt(jnp.max(jnp.sum(xl * xl, axis=1))) +
            jnp.sqrt(jnp.max(jnp.sum(xr * xr, axis=1))))
        m16a = jnp.full((16,), mshift, jnp.float32)
        if _USE_JNP_EDGE:
            v = xl[srcs] + xr[dsts]
            e = jnp.where(v > 0, v, 0.2 * v) @ att
            p = jnp.exp(e - mshift)
            denj = jax.ops.segment_sum(p, dsts, num_segments=NP)
            outj = jax.ops.segment_sum(p[:, None] * xl[srcs], dsts,
                                       num_segments=NP)
            raw = outj * jnp.where(denj > 0, 1.0 / denj, 0.0)[:, None]
        else:
            raw = _sc_gat(xl, xr, srcs, dsts, tb, att, m16a)
        sums, sqs, cnt = _stats(raw, batch3)
        x_g = _apply(raw, x_g, batch3, sums, sqs, cnt)

    return _pooled_head(x_g[:N], batch, fin_W, fin_b)


# speedup vs baseline: 7.8381x; 1.2413x over previous
import functools

import jax
import jax.numpy as jnp
from jax import lax
from jax.experimental import pallas as pl
from jax.experimental.pallas import tpu as pltpu
from jax.experimental.pallas import tpu_sc as plsc

_USE_JNP_EDGE = False
N = 10000
B = 100
D = 128
DOUT = 64
NT = 32
NPT = 320
NP = NT * NPT
E2 = 160000 + N
CE = 64
E2P = ((E2 + CE - 1) // CE) * CE
EPAD = E2P - E2
EBUF = 7168
NL = 16
RB = 1024
NB = NP // RB
HB = 1000
NHB = N // HB


def _splat_i(v):
    return jnp.full((NL,), v, jnp.int32)


def _sread(ref, i):
    return ref[pl.ds(i, 16)][0]


def _sc_body(xl_hbm, xr_hbm, srcs_hbm, dsts_hbm, tb_hbm, att_hbm, m_hbm,
             out_hbm, out_loc, den_loc, rows_v, rowsr_v, idx_v, dst_v, tb_v,
             tb_s, dst_s, att_loc, m_loc, sem0, sem1, sem2, sem3):
    cid = lax.axis_index("c")
    sid = lax.axis_index("s")
    wid = sid * 2 + cid
    n0 = wid * NPT

    pltpu.sync_copy(tb_hbm, tb_v)
    pltpu.sync_copy(att_hbm, att_loc)
    pltpu.sync_copy(m_hbm, m_loc)
    att8 = [att_loc[pl.ds(16 * k, 16)] for k in range(8)]
    m16 = m_loc[pl.ds(0, 16)]
    sems = [sem0, sem1]
    semsr = [sem2, sem3]

    for g in range(4):
        v = tb_v[pl.ds(16 * g, 16)]
        for l in range(16):
            tb_s[16 * g + l] = v[l]
    e0 = tb_s[wid]
    e1 = tb_s[wid + 1]
    c0 = e0 // CE
    base = c0 * CE
    nch = (e1 - base + CE - 1) // CE
    ng = (nch + 1) // 2
    GRP = CE // 16

    zero16 = jnp.zeros((16,), jnp.float32)

    def _zrow(i, _):
        for k in range(8):
            out_loc[i, pl.ds(16 * k, 16)] = zero16
        den_loc[i, pl.ds(0, 16)] = zero16
        return 0
    lax.fori_loop(0, NPT, _zrow, 0)

    def fetch_idx(t, b):
        off = base + t * CE
        pltpu.sync_copy(srcs_hbm.at[pl.ds(off, CE)], idx_v.at[b])
        pltpu.sync_copy(dsts_hbm.at[pl.ds(off, CE)], dst_v.at[b])

    def start_gather(b):
        pltpu.async_copy(xl_hbm.at[idx_v.at[b]], rows_v.at[b], sems[b])
        pltpu.async_copy(xr_hbm.at[dst_v.at[b]], rowsr_v.at[b], semsr[b])

    def wait_gather(b):
        pltpu.make_async_copy(xl_hbm.at[idx_v.at[b]], rows_v.at[b],
                              sems[b]).wait()
        pltpu.make_async_copy(xr_hbm.at[dst_v.at[b]], rowsr_v.at[b],
                              semsr[b]).wait()

    fetch_idx(0, 0)
    start_gather(0)

    def edge_body(cbase, b):
        def body(j, carry):
            jj = j - cbase
            dl = dst_s[b, jj] - n0
            acc = zero16
            rowk = []
            for k in range(8):
                a = rows_v[b, jj, pl.ds(16 * k, 16)]
                rowk.append(a)
                r = rowsr_v[b, jj, pl.ds(16 * k, 16)]
                s = a + r
                lr = jnp.maximum(s, 0.2 * s)
                acc = acc + lr * att8[k]
            e = jnp.sum(acc)
            p = jnp.exp(jnp.full((16,), e) - m16)
            den_loc[dl, pl.ds(0, 16)] = den_loc[dl, pl.ds(0, 16)] + p
            for k in range(8):
                out_loc[dl, pl.ds(16 * k, 16)] = (
                    out_loc[dl, pl.ds(16 * k, 16)] + p * rowk[k])
            return carry
        return body

    def group(g, carry):
        for b in range(2):
            t = g * 2 + b

            @pl.when(t < nch)
            def _():
                wait_gather(b)

                @pl.when(t + 1 < nch)
                def _():
                    fetch_idx(t + 1, 1 - b)
                    start_gather(1 - b)

                for q in range(CE // 16):
                    dl16 = dst_v[b, pl.ds(16 * q, 16)]
                    for l in range(16):
                        dst_s[b, 16 * q + l] = dl16[l]

            cbase = base + t * CE
            lo = jnp.maximum(e0, cbase)
            hi = jnp.minimum(e1, cbase + CE)
            carry = lax.fori_loop(lo, hi, edge_body(cbase, b), carry)
        return carry

    lax.fori_loop(0, ng, group, jnp.int32(0))

    def _nrow(i, _):
        denv = den_loc[i, pl.ds(0, 16)]
        rcp = jnp.where(denv > 0.0, 1.0 / denv, 0.0)
        for k in range(8):
            out_loc[i, pl.ds(16 * k, 16)] = out_loc[i, pl.ds(16 * k, 16)] * rcp
        return 0
    lax.fori_loop(0, NPT, _nrow, 0)

    pltpu.sync_copy(out_loc, out_hbm.at[pl.ds(n0, NPT)])


_sc_gat = functools.partial(
    pl.kernel,
    out_type=jax.ShapeDtypeStruct((NP, D), jnp.float32),
    mesh=plsc.VectorSubcoreMesh(core_axis_name="c", subcore_axis_name="s"),
    compiler_params=pltpu.CompilerParams(needs_layout_passes=False),
    scratch_types=[
        pltpu.VMEM((NPT, D), jnp.float32),
        pltpu.VMEM((NPT, 16), jnp.float32),
        pltpu.VMEM((2, CE, D), jnp.float32),
        pltpu.VMEM((2, CE, D), jnp.float32),
        pltpu.VMEM((2, CE), jnp.int32),
        pltpu.VMEM((2, CE), jnp.int32),
        pltpu.VMEM((64,), jnp.int32),
        pltpu.SMEM((64,), jnp.int32),
        pltpu.SMEM((2, CE), jnp.int32),
        pltpu.VMEM((D,), jnp.float32),
        pltpu.VMEM((16,), jnp.float32),
        pltpu.SemaphoreType.DMA,
        pltpu.SemaphoreType.DMA,
        pltpu.SemaphoreType.DMA,
        pltpu.SemaphoreType.DMA,
    ])(_sc_body)


def _mm2_kernel(x_ref, wl_ref, bl_ref, wr_ref, br_ref, xl_ref, xr_ref):
    xb = x_ref[...]
    xl_ref[...] = lax.dot_general(
        xb, wl_ref[...], (((1,), (1,)), ((), ())),
        preferred_element_type=jnp.float32) + bl_ref[...]
    xr_ref[...] = lax.dot_general(
        xb, wr_ref[...], (((1,), (1,)), ((), ())),
        preferred_element_type=jnp.float32) + br_ref[...]


def _mm2(x, Wl, bl, Wr, br):
    return pl.pallas_call(
        _mm2_kernel, grid=(NB,),
        in_specs=[
            pl.BlockSpec((RB, D), lambda i: (i, 0)),
            pl.BlockSpec((D, D), lambda i: (0, 0)),
            pl.BlockSpec((1, D), lambda i: (0, 0)),
            pl.BlockSpec((D, D), lambda i: (0, 0)),
            pl.BlockSpec((1, D), lambda i: (0, 0)),
        ],
        out_specs=[pl.BlockSpec((RB, D), lambda i: (i, 0))] * 2,
        out_shape=[jax.ShapeDtypeStruct((NP, D), jnp.float32)] * 2,
    )(x, Wl, bl.reshape(1, D), Wr, br.reshape(1, D))


def _stats_kernel(h_ref, batch_ref, sums_ref, sqs_ref, cnt_ref,
                  acc_s, acc_q, acc_c):
    i = pl.program_id(0)

    @pl.when(i == 0)
    def _():
        acc_s[...] = jnp.zeros_like(acc_s)
        acc_q[...] = jnp.zeros_like(acc_q)
        acc_c[...] = jnp.zeros_like(acc_c)

    hb = h_ref[...]
    bv = batch_ref[0, 0, :]
    oh = (bv[:, None] ==
          lax.broadcasted_iota(jnp.int32, (RB, B), 1)).astype(jnp.float32)
    acc_s[...] += lax.dot_general(oh, hb, (((0,), (0,)), ((), ())),
                                  preferred_element_type=jnp.float32)
    acc_q[...] += lax.dot_general(oh, hb * hb, (((0,), (0,)), ((), ())),
                                  preferred_element_type=jnp.float32)
    acc_c[...] += jnp.sum(oh, axis=0)[None, :]

    @pl.when(i == NB - 1)
    def _():
        sums_ref[...] = acc_s[...]
        sqs_ref[...] = acc_q[...]
        cnt_ref[...] = acc_c[...]


def _stats(h, batch3):
    return pl.pallas_call(
        _stats_kernel, grid=(NB,),
        in_specs=[
            pl.BlockSpec((RB, D), lambda i: (i, 0)),
            pl.BlockSpec((1, 1, RB), lambda i: (i, 0, 0)),
        ],
        out_specs=[
            pl.BlockSpec((B, D), lambda i: (0, 0)),
            pl.BlockSpec((B, D), lambda i: (0, 0)),
            pl.BlockSpec((1, B), lambda i: (0, 0)),
        ],
        out_shape=[
            jax.ShapeDtypeStruct((B, D), jnp.float32),
            jax.ShapeDtypeStruct((B, D), jnp.float32),
            jax.ShapeDtypeStruct((1, B), jnp.float32),
        ],
        scratch_shapes=[
            pltpu.VMEM((B, D), jnp.float32),
            pltpu.VMEM((B, D), jnp.float32),
            pltpu.VMEM((1, B), jnp.float32),
        ],
    )(h, batch3)


def _apply_kernel(h_ref, xp_ref, batch_ref, sums_ref, sqs_ref, cnt_ref,
                  out_ref):
    bv = batch_ref[0, 0, :]
    oh = (bv[:, None] ==
          lax.broadcasted_iota(jnp.int32, (RB, B), 1)).astype(jnp.float32)
    cnt = jnp.maximum(cnt_ref[0, :], 1.0)
    mean = sums_ref[...] / cnt[:, None]
    var = sqs_ref[...] / cnt[:, None] - mean * mean
    meanrow = lax.dot_general(oh, mean, (((1,), (0,)), ((), ())),
                              preferred_element_type=jnp.float32)
    varrow = lax.dot_general(oh, var, (((1,), (0,)), ((), ())),
                             preferred_element_type=jnp.float32)
    xc = h_ref[...] - meanrow
    hn = xc * lax.rsqrt(varrow + 1e-5)
    out_ref[...] = jnp.where(bv[:, None] < B,
                             jnp.maximum(hn, 0.0) + xp_ref[...], 0.0)


def _apply(h, xprev, batch3, sums, sqs, cnt):
    return pl.pallas_call(
        _apply_kernel, grid=(NB,),
        in_specs=[
            pl.BlockSpec((RB, D), lambda i: (i, 0)),
            pl.BlockSpec((RB, D), lambda i: (i, 0)),
            pl.BlockSpec((1, 1, RB), lambda i: (i, 0, 0)),
            pl.BlockSpec((B, D), lambda i: (0, 0)),
            pl.BlockSpec((B, D), lambda i: (0, 0)),
            pl.BlockSpec((1, B), lambda i: (0, 0)),
        ],
        out_specs=pl.BlockSpec((RB, D), lambda i: (i, 0)),
        out_shape=jax.ShapeDtypeStruct((NP, D), jnp.float32),
    )(h, xprev, batch3, sums, sqs, cnt)


def _head_kernel(x_ref, batch_ref, w_ref, b_ref, out_ref, acc_ref):
    i = pl.program_id(0)

    @pl.when(i == 0)
    def _():
        acc_ref[...] = jnp.zeros_like(acc_ref)

    rows = x_ref[...]
    bvals = batch_ref[0, 0, :]
    onehot = (bvals[:, None] ==
              lax.broadcasted_iota(jnp.int32, (HB, B), 1)).astype(jnp.float32)
    acc_ref[...] += lax.dot_general(onehot, rows, (((0,), (0,)), ((), ())),
                                    preferred_element_type=jnp.float32)

    @pl.when(i == NHB - 1)
    def _():
        out_ref[...] = lax.dot_general(
            acc_ref[...], w_ref[...], (((1,), (1,)), ((), ())),
            preferred_element_type=jnp.float32) + b_ref[...]


def _pooled_head(x_g, batch, fin_W, fin_b):
    batch3 = batch.reshape(NHB, 1, HB)
    return pl.pallas_call(
        _head_kernel,
        grid=(NHB,),
        in_specs=[
            pl.BlockSpec((HB, D), lambda i: (i, 0)),
            pl.BlockSpec((1, 1, HB), lambda i: (i, 0, 0)),
            pl.BlockSpec((DOUT, D), lambda i: (0, 0)),
            pl.BlockSpec((1, DOUT), lambda i: (0, 0)),
        ],
        out_specs=pl.BlockSpec((B, DOUT), lambda i: (0, 0)),
        out_shape=jax.ShapeDtypeStruct((B, DOUT), jnp.float32),
        scratch_shapes=[pltpu.VMEM((B, D), jnp.float32)],
    )(x_g, batch3, fin_W, fin_b.reshape(1, DOUT))


def kernel(x, edge_indices, batch, g0_Wl, g0_bl, g0_Wr, g0_br, g0_att,
           g0_bias, g1_Wl, g1_bl, g1_Wr, g1_br, g1_att, g1_bias,
           dp0_W, dp0_b, dp1_W, dp1_b, fin_W, fin_b):
    x_pad = jnp.concatenate(
        [x, jnp.zeros((NP - N, D), jnp.float32)], axis=0)
    batch_pad = jnp.concatenate(
        [batch.astype(jnp.int32), jnp.full((NP - N,), B, jnp.int32)])
    batch3 = batch_pad.reshape(NB, 1, RB)

    loop = jnp.arange(N, dtype=jnp.int32)
    ei = edge_indices[0]
    srcp = jnp.concatenate([ei[0].astype(jnp.int32), loop,
                            jnp.zeros((EPAD,), jnp.int32)])
    dstp = jnp.concatenate([ei[1].astype(jnp.int32), loop,
                            jnp.full((EPAD,), NP - 1, jnp.int32)])
    skey = jnp.sort((dstp << 14) | srcp)
    srcs = skey & (16384 - 1)
    dsts = skey >> 14
    tb = jnp.searchsorted(
        dsts, jnp.arange(NT + 1, dtype=jnp.int32) * NPT).astype(jnp.int32)
    tb = jnp.concatenate([tb, jnp.zeros((64 - NT - 1,), jnp.int32)])

    x_g = x_pad
    for (Wl, bl, Wr, br, att) in (
            (g0_Wl, g0_bl, g0_Wr, g0_br, g0_att),
            (g1_Wl, g1_bl, g1_Wr, g1_br, g1_att)):
        xl, xr = _mm2(x_g, Wl, bl, Wr, br)
        mshift = jnp.sqrt(jnp.dot(att, att)) * (
            jnp.sqrt(jnp.max(jnp.sum(xl * xl, axis=1))) +
            jnp.sqrt(jnp.max(jnp.sum(xr * xr, axis=1))))
        m16a = jnp.full((16,), mshift, jnp.float32)
        if _USE_JNP_EDGE:
            v = xl[srcs] + xr[dsts]
            e = jnp.where(v > 0, v, 0.2 * v) @ att
            p = jnp.exp(e - mshift)
            denj = jax.ops.segment_sum(p, dsts, num_segments=NP)
            outj = jax.ops.segment_sum(p[:, None] * xl[srcs], dsts,
                                       num_segments=NP)
            raw = outj * jnp.where(denj > 0, 1.0 / denj, 0.0)[:, None]
        else:
            raw = _sc_gat(xl, xr, srcs, dsts, tb, att, m16a)
        sums, sqs, cnt = _stats(raw, batch3)
        x_g = _apply(raw, x_g, batch3, sums, sqs, cnt)

    return _pooled_head(x_g[:N], batch, fin_W, fin_b)
